# Initial kernel scaffold; baseline (speedup 1.0000x reference)
import jax
import jax.numpy as jnp
from jax.experimental import pallas as pl


def kernel(dynamic_ids, dynamic_lengths, table):
    raise NotImplementedError("write your pallas kernel here")


# speedup vs baseline: 7.9202x; 7.9202x over previous
import jax
import jax.numpy as jnp
from jax import lax
from jax.experimental import pallas as pl
from jax.experimental.pallas import tpu as pltpu
from jax.experimental.pallas import tpu_sc as plsc

NC = 2
NS = 16
NW = NC * NS
LANES = 16
IB = 8
GB = 4


def _build(B, H, V, D, n_chunks, chunk):
  rpt = B // NW
  groups = rpt // LANES

  mesh = plsc.VectorSubcoreMesh(
      core_axis_name="c", subcore_axis_name="s",
      num_cores=NC, num_subcores=NS)

  def body(table_hbm, ids_hbm, len_hbm, out_hbm,
           ids_v, len_v, rows_v, out_v, *sems):
    ids_sems = sems[0:IB]
    row_sems = sems[IB:IB + GB]
    wid = lax.axis_index("s") * NC + lax.axis_index("c")
    base = wid * rpt

    pltpu.sync_copy(len_hbm.at[pl.ds(wid * groups, groups)], len_v)

    def ids_copy(row, slot):
      return pltpu.make_async_copy(
          ids_hbm.at[base + row], ids_v.at[slot], ids_sems[slot])

    def gather_copy(row_slot, c, g_slot):
      return pltpu.make_async_copy(
          table_hbm.at[ids_v.at[row_slot, c]], rows_v.at[g_slot, c],
          row_sems[g_slot])

    for q in range(IB):
      ids_copy(q, q).start()
    for q in range(GB):
      ids_copy(q, q).wait()
      for c in range(n_chunks):
        gather_copy(q, c, q).start()

    @pl.loop(0, groups)
    def _(g):
      lv = len_v[g, :]
      for r in range(LANES):
        row = g * LANES + r
        gs = r % GB
        for c in range(n_chunks):
          gather_copy(r % IB, c, gs).wait()
        accs = [None] * 4
        k = 0
        for c in range(n_chunks):
          for j in range(chunk):
            v = rows_v[gs, c, j, :]
            accs[k % 4] = v if accs[k % 4] is None else accs[k % 4] + v
            k += 1
        acc = (accs[0] + accs[1]) + (accs[2] + accs[3])
        out_v[row, :] = acc / jnp.full((LANES,), lv[r])

        @pl.when(row + IB < rpt)
        def _():
          ids_copy(row + IB, r % IB).start()

        @pl.when(row + GB < rpt)
        def _():
          ids_copy(row + GB, (r + GB) % IB).wait()
          for c in range(n_chunks):
            gather_copy((r + GB) % IB, c, gs).start()

    pltpu.sync_copy(out_v, out_hbm.at[pl.ds(base, rpt)])

  return pl.kernel(
      body,
      out_type=jax.ShapeDtypeStruct((B, D), jnp.float32),
      mesh=mesh,
      scratch_types=[
          pltpu.VMEM((IB, n_chunks, chunk), jnp.int32),
          pltpu.VMEM((groups, LANES), jnp.float32),
          pltpu.VMEM((GB, n_chunks, chunk, D), jnp.float32),
          pltpu.VMEM((rpt, D), jnp.float32),
      ] + [pltpu.SemaphoreType.DMA] * (IB + GB),
      compiler_params=pltpu.CompilerParams(use_tc_tiling_on_sc=False),
  )


def kernel(dynamic_ids, dynamic_lengths, table):
  B, H = dynamic_ids.shape
  V, D = table.shape
  assert D == LANES and B % (NW * LANES) == 0
  n_chunks = 2
  chunk = H // n_chunks
  assert chunk * n_chunks == H and chunk <= 128 and H % 4 == 0

  ids = dynamic_ids.astype(jnp.int32).reshape(B, n_chunks, chunk)
  lens = dynamic_lengths.astype(jnp.float32).reshape(B // LANES, LANES)
  fn = _build(B, H, V, D, n_chunks, chunk)
  return fn(table, ids, lens)


# speedup vs baseline: 8.8986x; 1.1235x over previous
import jax
import jax.numpy as jnp
from jax import lax
from jax.experimental import pallas as pl
from jax.experimental.pallas import tpu as pltpu
from jax.experimental.pallas import tpu_sc as plsc

NC = 2
NS = 16
NW = NC * NS
LANES = 16
IB = 8
GB = 4


def _build(B, H, V, D, n_chunks, chunk):
  rpt = B // NW
  groups = rpt // LANES

  mesh = plsc.VectorSubcoreMesh(
      core_axis_name="c", subcore_axis_name="s",
      num_cores=NC, num_subcores=NS)

  def body(table_hbm, ids_hbm, len_hbm, out_hbm,
           ids_v, len_v, rows_v, out_v, *sems):
    ids_sems = sems[0:IB]
    row_sems = sems[IB:IB + GB]
    wid = lax.axis_index("s") * NC + lax.axis_index("c")
    base = wid * rpt

    pltpu.sync_copy(len_hbm.at[pl.ds(wid * groups, groups)], len_v)

    def ids_copy(row, slot):
      return pltpu.make_async_copy(
          ids_hbm.at[base + row], ids_v.at[slot], ids_sems[slot])

    def start_gather(row_slot, c, g_slot):
      pltpu.async_copy(table_hbm.at[ids_v.at[row_slot, c]],
                       rows_v.at[g_slot], row_sems[g_slot], add=True)

    def wait_gather(row_slot, c, g_slot):
      pltpu.make_async_copy(table_hbm.at[ids_v.at[row_slot, c]],
                            rows_v.at[g_slot], row_sems[g_slot]).wait()

    zero = jnp.zeros((LANES,), jnp.float32)

    def zero_slot(g_slot):
      for j in range(chunk):
        rows_v[g_slot, j, :] = zero

    for q in range(IB):
      ids_copy(q, q).start()
    for q in range(GB):
      zero_slot(q)
      ids_copy(q, q).wait()
      for c in range(n_chunks):
        start_gather(q, c, q)

    @pl.loop(0, groups)
    def _(g):
      lv = len_v[g, :]
      for r in range(LANES):
        row = g * LANES + r
        gs = r % GB
        for c in range(n_chunks):
          wait_gather(r % IB, c, gs)
        accs = [None] * 4
        for j in range(chunk):
          v = rows_v[gs, j, :]
          accs[j % 4] = v if accs[j % 4] is None else accs[j % 4] + v
        acc = (accs[0] + accs[1]) + (accs[2] + accs[3])
        out_v[row, :] = acc / jnp.full((LANES,), lv[r])
        zero_slot(gs)

        @pl.when(row + IB < rpt)
        def _():
          ids_copy(row + IB, r % IB).start()

        @pl.when(row + GB < rpt)
        def _():
          ids_copy(row + GB, (r + GB) % IB).wait()
          for c in range(n_chunks):
            start_gather((r + GB) % IB, c, gs)

    pltpu.sync_copy(out_v, out_hbm.at[pl.ds(base, rpt)])

  return pl.kernel(
      body,
      out_type=jax.ShapeDtypeStruct((B, D), jnp.float32),
      mesh=mesh,
      scratch_types=[
          pltpu.VMEM((IB, n_chunks, chunk), jnp.int32),
          pltpu.VMEM((groups, LANES), jnp.float32),
          pltpu.VMEM((GB, chunk, D), jnp.float32),
          pltpu.VMEM((rpt, D), jnp.float32),
      ] + [pltpu.SemaphoreType.DMA] * (IB + GB),
      compiler_params=pltpu.CompilerParams(use_tc_tiling_on_sc=False),
  )


def kernel(dynamic_ids, dynamic_lengths, table):
  B, H = dynamic_ids.shape
  V, D = table.shape
  assert D == LANES and B % (NW * LANES) == 0
  n_chunks = 4
  chunk = H // n_chunks
  assert chunk * n_chunks == H and chunk <= 128 and chunk >= 4

  ids = dynamic_ids.astype(jnp.int32).reshape(B, n_chunks, chunk)
  lens = dynamic_lengths.astype(jnp.float32).reshape(B // LANES, LANES)
  fn = _build(B, H, V, D, n_chunks, chunk)
  return fn(table, ids, lens)


# speedup vs baseline: 11.5027x; 1.2926x over previous
import jax
import jax.numpy as jnp
from jax import lax
from jax.experimental import pallas as pl
from jax.experimental.pallas import tpu as pltpu
from jax.experimental.pallas import tpu_sc as plsc

NC = 2
NS = 16
NW = NC * NS
LANES = 16
IB = 8
GB = 4


def _build(B, H, V, D, n_chunks, chunk):
  rpt = B // NW
  groups = rpt // LANES

  mesh = plsc.VectorSubcoreMesh(
      core_axis_name="c", subcore_axis_name="s",
      num_cores=NC, num_subcores=NS)

  def body(table_hbm, ids_hbm, len_hbm, out_hbm,
           ids_v, len_v, rows_v, out_v, *sems):
    ids_sems = sems[0:IB]
    row_sems = sems[IB:IB + GB]
    wid = lax.axis_index("s") * NC + lax.axis_index("c")
    base = wid * rpt

    pltpu.sync_copy(len_hbm.at[pl.ds(wid * groups, groups)], len_v)

    def ids_copy(row, slot):
      return pltpu.make_async_copy(
          ids_hbm.at[base + row], ids_v.at[slot], ids_sems[slot])

    def ids_chunk(slot, c):
      return ids_v.at[slot, pl.ds(c * chunk, chunk)]

    def start_gather(row_slot, c, g_slot):
      pltpu.async_copy(table_hbm.at[ids_chunk(row_slot, c)],
                       rows_v.at[g_slot], row_sems[g_slot], add=True)

    def wait_gather(row_slot, c, g_slot):
      pltpu.make_async_copy(table_hbm.at[ids_chunk(row_slot, c)],
                            rows_v.at[g_slot], row_sems[g_slot]).wait()

    zero = jnp.zeros((LANES,), jnp.float32)

    def zero_slot(g_slot):
      for j in range(chunk):
        rows_v[g_slot, j, :] = zero

    for q in range(IB):
      ids_copy(q, q).start()
    for q in range(GB):
      zero_slot(q)
      ids_copy(q, q).wait()
      for c in range(n_chunks):
        start_gather(q, c, q)

    @pl.loop(0, groups)
    def _(g):
      lv = len_v[g, :]
      for r in range(LANES):
        row = g * LANES + r
        gs = r % GB
        for c in range(n_chunks):
          wait_gather(r % IB, c, gs)
        accs = [None] * 4
        for j in range(chunk):
          v = rows_v[gs, j, :]
          accs[j % 4] = v if accs[j % 4] is None else accs[j % 4] + v
        acc = (accs[0] + accs[1]) + (accs[2] + accs[3])
        out_v[row, :] = acc / jnp.full((LANES,), lv[r])
        zero_slot(gs)

        @pl.when(row + IB < rpt)
        def _():
          ids_copy(row + IB, r % IB).start()

        @pl.when(row + GB < rpt)
        def _():
          ids_copy(row + GB, (r + GB) % IB).wait()
          for c in range(n_chunks):
            start_gather((r + GB) % IB, c, gs)

    pltpu.sync_copy(out_v, out_hbm.at[pl.ds(base, rpt)])

  return pl.kernel(
      body,
      out_type=jax.ShapeDtypeStruct((B, D), jnp.float32),
      mesh=mesh,
      scratch_types=[
          pltpu.VMEM((IB, H), jnp.int32),
          pltpu.VMEM((groups, LANES), jnp.float32),
          pltpu.VMEM((GB, chunk, D), jnp.float32),
          pltpu.VMEM((rpt, D), jnp.float32),
      ] + [pltpu.SemaphoreType.DMA] * (IB + GB),
      compiler_params=pltpu.CompilerParams(use_tc_tiling_on_sc=False),
  )


def _detile_table(table, V, D):
  C = 4096

  def body(x_ref, o_ref):
    t = x_ref[...].T.reshape(C // 8, 8, D)
    o_ref[...] = jnp.concatenate([t[:, k, :] for k in range(8)], axis=1)

  tab_t = table.T
  grid = (V + C - 1) // C
  out = pl.pallas_call(
      body,
      grid=(grid,),
      in_specs=[pl.BlockSpec((D, C), lambda i: (0, i))],
      out_specs=pl.BlockSpec((C // 8, 8 * D), lambda i: (i, 0)),
      out_shape=jax.ShapeDtypeStruct((V // 8, 8 * D), jnp.float32),
  )(tab_t)
  return out.reshape(V, D)


def kernel(dynamic_ids, dynamic_lengths, table):
  B, H = dynamic_ids.shape
  V, D = table.shape
  assert D == LANES and B % (NW * LANES) == 0 and V % 8 == 0
  n_chunks = 5
  chunk = H // n_chunks
  assert chunk * n_chunks == H and chunk <= 128 and chunk >= 4
  assert chunk % 8 == 0

  ids = dynamic_ids.astype(jnp.int32)
  lens = dynamic_lengths.astype(jnp.float32).reshape(B // LANES, LANES)
  tab = _detile_table(table, V, D)
  fn = _build(B, H, V, D, n_chunks, chunk)
  return fn(tab, ids, lens)


# speedup vs baseline: 11.9414x; 1.0381x over previous
import jax
import jax.numpy as jnp
from jax import lax
from jax.experimental import pallas as pl
from jax.experimental.pallas import tpu as pltpu
from jax.experimental.pallas import tpu_sc as plsc

NC = 2
NS = 16
NW = NC * NS
LANES = 16
IB = 8
GB = 4


def _build(B, H, V, D, n_chunks, chunk):
  rpt = B // NW
  groups = rpt // LANES

  mesh = plsc.VectorSubcoreMesh(
      core_axis_name="c", subcore_axis_name="s",
      num_cores=NC, num_subcores=NS)

  def body(table_hbm, ids_hbm, len_hbm, out_hbm,
           ids_v, len_v, rows_v, out_v, *sems):
    ids_sems = sems[0:IB]
    row_sems = sems[IB:IB + GB]
    wid = lax.axis_index("s") * NC + lax.axis_index("c")
    base = wid * rpt

    pltpu.sync_copy(len_hbm.at[pl.ds(wid * groups, groups)], len_v)

    def ids_copy(row, slot):
      return pltpu.make_async_copy(
          ids_hbm.at[base + row], ids_v.at[slot], ids_sems[slot])

    def ids_chunk(slot, c):
      return ids_v.at[slot, pl.ds(c * chunk, chunk)]

    def start_gather(row_slot, c, g_slot):
      pltpu.async_copy(table_hbm.at[ids_chunk(row_slot, c)],
                       rows_v.at[g_slot], row_sems[g_slot], add=True)

    def wait_gather(row_slot, c, g_slot):
      pltpu.make_async_copy(table_hbm.at[ids_chunk(row_slot, c)],
                            rows_v.at[g_slot], row_sems[g_slot]).wait()

    zero = jnp.zeros((LANES,), jnp.float32)

    def zero_slot(g_slot):
      for j in range(chunk):
        rows_v[g_slot, j, :] = zero

    for q in range(IB):
      ids_copy(q, q).start()
    for q in range(GB):
      zero_slot(q)
      ids_copy(q, q).wait()
      for c in range(n_chunks):
        start_gather(q, c, q)

    @pl.loop(0, groups)
    def _(g):
      lv = len_v[g, :]
      for r in range(LANES):
        row = g * LANES + r
        gs = r % GB
        for c in range(n_chunks):
          wait_gather(r % IB, c, gs)
        accs = [None] * 4
        for j in range(chunk):
          v = rows_v[gs, j, :]
          accs[j % 4] = v if accs[j % 4] is None else accs[j % 4] + v
        acc = (accs[0] + accs[1]) + (accs[2] + accs[3])
        out_v[row, :] = acc / jnp.full((LANES,), lv[r])
        zero_slot(gs)

        @pl.when(row + IB < rpt)
        def _():
          ids_copy(row + IB, r % IB).start()

        @pl.when(row + GB < rpt)
        def _():
          ids_copy(row + GB, (r + GB) % IB).wait()
          for c in range(n_chunks):
            start_gather((r + GB) % IB, c, gs)

    pltpu.sync_copy(out_v, out_hbm.at[pl.ds(base, rpt)])

  return pl.kernel(
      body,
      out_type=jax.ShapeDtypeStruct((B, D), jnp.float32),
      mesh=mesh,
      scratch_types=[
          pltpu.VMEM((IB, H), jnp.int32),
          pltpu.VMEM((groups, LANES), jnp.float32),
          pltpu.VMEM((GB, chunk, D), jnp.float32),
          pltpu.VMEM((rpt, D), jnp.float32),
      ] + [pltpu.SemaphoreType.DMA] * (IB + GB),
      compiler_params=pltpu.CompilerParams(use_tc_tiling_on_sc=False),
  )


C_BLK = 8192


def _detile_table(table, V, D):
  grid = (V + C_BLK - 1) // C_BLK
  vp = grid * C_BLK

  def body(x_ref, o_ref):
    t = x_ref[...].T
    s = C_BLK // 8
    for k in range(8):
      o_ref[:, pl.ds(k * D, D)] = t[k * s:(k + 1) * s, :]

  tab_t = table.T
  out = pl.pallas_call(
      body,
      grid=(grid,),
      in_specs=[pl.BlockSpec((D, C_BLK), lambda i: (0, i))],
      out_specs=pl.BlockSpec((C_BLK // 8, 8 * D), lambda i: (i, 0)),
      out_shape=jax.ShapeDtypeStruct((vp // 8, 8 * D), jnp.float32),
  )(tab_t)
  return out.reshape(vp, D)


def kernel(dynamic_ids, dynamic_lengths, table):
  B, H = dynamic_ids.shape
  V, D = table.shape
  assert D == LANES and B % (NW * LANES) == 0 and V % 8 == 0
  n_chunks = 5
  chunk = H // n_chunks
  assert chunk * n_chunks == H and chunk <= 128 and chunk >= 4
  assert chunk % 8 == 0

  m = dynamic_ids.astype(jnp.int32)
  s = C_BLK // 8
  ids = (m // C_BLK) * C_BLK + (m % s) * 8 + (m % C_BLK) // s
  lens = dynamic_lengths.astype(jnp.float32).reshape(B // LANES, LANES)
  tab = _detile_table(table, V, D)
  fn = _build(B, H, tab.shape[0], D, n_chunks, chunk)
  return fn(tab, ids, lens)


# speedup vs baseline: 12.3666x; 1.0356x over previous
import jax
import jax.numpy as jnp
from jax import lax
from jax.experimental import pallas as pl
from jax.experimental.pallas import tpu as pltpu
from jax.experimental.pallas import tpu_sc as plsc

NC = 2
NS = 16
NW = NC * NS
LANES = 16
IB = 16
GB = 8


def _build(B, H, V, D, n_chunks, chunk):
  rpt = B // NW
  groups = rpt // LANES

  mesh = plsc.VectorSubcoreMesh(
      core_axis_name="c", subcore_axis_name="s",
      num_cores=NC, num_subcores=NS)

  def body(table_hbm, ids_hbm, len_hbm, out_hbm,
           ids_v, len_v, rows_v, out_v, *sems):
    ids_sems = sems[0:IB]
    row_sems = sems[IB:IB + GB]
    wid = lax.axis_index("s") * NC + lax.axis_index("c")
    base = wid * rpt

    pltpu.sync_copy(len_hbm.at[pl.ds(wid * groups, groups)], len_v)

    def ids_copy(row, slot):
      return pltpu.make_async_copy(
          ids_hbm.at[base + row], ids_v.at[slot], ids_sems[slot])

    def ids_chunk(slot, c):
      return ids_v.at[slot, pl.ds(c * chunk, chunk)]

    def start_gather(row_slot, c, g_slot):
      pltpu.async_copy(table_hbm.at[ids_chunk(row_slot, c)],
                       rows_v.at[g_slot], row_sems[g_slot], add=True)

    def wait_gather(row_slot, c, g_slot):
      pltpu.make_async_copy(table_hbm.at[ids_chunk(row_slot, c)],
                            rows_v.at[g_slot], row_sems[g_slot]).wait()

    zero = jnp.zeros((LANES,), jnp.float32)

    def zero_slot(g_slot):
      for j in range(chunk):
        rows_v[g_slot, j, :] = zero

    for q in range(IB):
      ids_copy(q, q).start()
    for q in range(GB):
      zero_slot(q)
      ids_copy(q, q).wait()
      for c in range(n_chunks):
        start_gather(q, c, q)

    @pl.loop(0, groups)
    def _(g):
      lv = len_v[g, :]
      for r in range(LANES):
        row = g * LANES + r
        gs = r % GB
        for c in range(n_chunks):
          wait_gather(r % IB, c, gs)
        accs = [None] * 4
        for j in range(chunk):
          v = rows_v[gs, j, :]
          accs[j % 4] = v if accs[j % 4] is None else accs[j % 4] + v
        acc = (accs[0] + accs[1]) + (accs[2] + accs[3])
        out_v[row, :] = acc / jnp.full((LANES,), lv[r])
        zero_slot(gs)

        @pl.when(row + IB < rpt)
        def _():
          ids_copy(row + IB, r % IB).start()

        @pl.when(row + GB < rpt)
        def _():
          ids_copy(row + GB, (r + GB) % IB).wait()
          for c in range(n_chunks):
            start_gather((r + GB) % IB, c, gs)

    pltpu.sync_copy(out_v, out_hbm.at[pl.ds(base, rpt)])

  return pl.kernel(
      body,
      out_type=jax.ShapeDtypeStruct((B, D), jnp.float32),
      mesh=mesh,
      scratch_types=[
          pltpu.VMEM((IB, H), jnp.int32),
          pltpu.VMEM((groups, LANES), jnp.float32),
          pltpu.VMEM((GB, chunk, D), jnp.float32),
          pltpu.VMEM((rpt, D), jnp.float32),
      ] + [pltpu.SemaphoreType.DMA] * (IB + GB),
      compiler_params=pltpu.CompilerParams(use_tc_tiling_on_sc=False),
  )


C_BLK = 8192


def _detile_table(table, V, D):
  grid = (V + C_BLK - 1) // C_BLK
  vp = grid * C_BLK

  def body(x_ref, o_ref):
    t = x_ref[...].T
    s = C_BLK // 8
    for k in range(8):
      o_ref[:, pl.ds(k * D, D)] = t[k * s:(k + 1) * s, :]

  tab_t = table.T
  out = pl.pallas_call(
      body,
      grid=(grid,),
      in_specs=[pl.BlockSpec((D, C_BLK), lambda i: (0, i))],
      out_specs=pl.BlockSpec((C_BLK // 8, 8 * D), lambda i: (i, 0)),
      out_shape=jax.ShapeDtypeStruct((vp // 8, 8 * D), jnp.float32),
  )(tab_t)
  return out.reshape(vp, D)


def kernel(dynamic_ids, dynamic_lengths, table):
  B, H = dynamic_ids.shape
  V, D = table.shape
  assert D == LANES and B % (NW * LANES) == 0 and V % 8 == 0
  n_chunks = 5
  chunk = H // n_chunks
  assert chunk * n_chunks == H and chunk <= 128 and chunk >= 4
  assert chunk % 8 == 0

  m = dynamic_ids.astype(jnp.int32)
  s = C_BLK // 8
  ids = (m // C_BLK) * C_BLK + (m % s) * 8 + (m % C_BLK) // s
  lens = dynamic_lengths.astype(jnp.float32).reshape(B // LANES, LANES)
  tab = _detile_table(table, V, D)
  fn = _build(B, H, tab.shape[0], D, n_chunks, chunk)
  return fn(tab, ids, lens)
